# Initial kernel scaffold; baseline (speedup 1.0000x reference)
import jax
import jax.numpy as jnp
from jax.experimental import pallas as pl


def kernel(data, emb, lin_W, att_i, att_j, att_em_i, att_em_j, gnn_bias, bn1_g, bn1_b, bn2_g, bn2_b, out_W, out_b):
    raise NotImplementedError("write your pallas kernel here")


# speedup vs baseline: 24.7738x; 24.7738x over previous
import jax
import jax.numpy as jnp
from jax.experimental import pallas as pl
from jax.experimental.pallas import tpu as pltpu

_NEG_SLOPE = 0.2
_EPS = 1e-5
_TOPK = 32


def _pick_block(n, want):
    if n % want == 0:
        return want
    return n


def _k1_body(data, emb, linw, ai, aj, aei, aej, xlin, s, t, invn):
    e = emb[...]
    w = linw[...]
    bsz = data.shape[0]
    xs, ss, ts = [], [], []
    for b in range(bsz):
        xb = jax.lax.dot_general(data[b], w, (((1,), (0,)), ((), ())))
        xs.append(xb)
        ss.append(jnp.sum(xb * ai[...], axis=-1) + jnp.sum(e * aei[...], axis=-1))
        ts.append(jnp.sum(xb * aj[...], axis=-1) + jnp.sum(e * aej[...], axis=-1))
    xlin[...] = jnp.stack(xs)
    s[...] = jnp.stack(ss)[None]
    t[...] = jnp.stack(ts)[None]
    invn[...] = jax.lax.rsqrt(jnp.sum(e * e, axis=-1))[None, None, :]


def _k2_body(embi, emba, invn, s, t, xlin, msgs, sums, sumsq, vals, *, ib, n, topk):
    i0 = pl.program_id(0) * ib
    bsz = t.shape[0]
    score = jax.lax.dot_general(embi[...], emba[...], (((1,), (1,)), ((), ())))
    score = score * invn[...]
    vals[...] = score
    neg = jnp.float32(-jnp.inf)

    def extract(_, carry):
        v = vals[...]
        m = jnp.max(v, axis=1, keepdims=True)
        vals[...] = jnp.where(v == m, neg, v)
        return carry

    jax.lax.fori_loop(0, topk, extract, 0)
    topk_mask = vals[...] == neg
    rows = jax.lax.broadcasted_iota(jnp.int32, (ib, n), 0) + i0
    cols = jax.lax.broadcasted_iota(jnp.int32, (ib, n), 1)
    allowed = jnp.logical_or(topk_mask, rows == cols)

    ms = []
    for b in range(bsz):
        sb = s[0, b]
        alpha = sb[:, None] + t[b, :][None, :]
        alpha = jnp.where(alpha > 0, alpha, _NEG_SLOPE * alpha)
        alpha = jnp.where(allowed, alpha, neg)
        mx = jnp.max(alpha, axis=1, keepdims=True)
        ex = jnp.exp(alpha - mx)
        den = jnp.sum(ex, axis=1, keepdims=True)
        a = ex / den
        ms.append(
            jax.lax.dot_general(
                a, xlin[b], (((1,), (0,)), ((), ())),
                precision=jax.lax.Precision.HIGHEST,
            )
        )
    mall = jnp.stack(ms)
    msgs[...] = mall
    ps = jnp.sum(jnp.sum(mall, axis=0), axis=0)[None, :]
    pq = jnp.sum(jnp.sum(mall * mall, axis=0), axis=0)[None, :]

    @pl.when(pl.program_id(0) == 0)
    def _init():
        sums[...] = jnp.zeros_like(sums)
        sumsq[...] = jnp.zeros_like(sumsq)

    sums[...] += ps
    sumsq[...] += pq


def _k3_body(msgs, emb, bias, g1, b1, sums, sumsq, y, ys, yq, *, cnt):
    m0 = sums[...] / cnt
    v1 = sumsq[...] / cnt - m0 * m0
    m1 = m0 + bias[...]
    inv1 = jax.lax.rsqrt(v1 + _EPS)
    e = emb[...]
    bsz = msgs.shape[0]
    outs = []
    for b in range(bsz):
        o = msgs[b] + bias[...]
        o = (o - m1) * inv1 * g1[...] + b1[...]
        o = jnp.maximum(o, 0.0)
        outs.append(o * e)
    yall = jnp.stack(outs)
    y[...] = yall
    ps = jnp.sum(jnp.sum(yall, axis=0), axis=0)[None, :]
    pq = jnp.sum(jnp.sum(yall * yall, axis=0), axis=0)[None, :]

    @pl.when(pl.program_id(0) == 0)
    def _init():
        ys[...] = jnp.zeros_like(ys)
        yq[...] = jnp.zeros_like(yq)

    ys[...] += ps
    yq[...] += pq


def _k4_body(y, g2, b2, ys, yq, ow, ob, out, *, cnt):
    m2 = ys[...] / cnt
    v2 = yq[...] / cnt - m2 * m2
    inv2 = jax.lax.rsqrt(v2 + _EPS)
    bsz = y.shape[0]
    rows = []
    for b in range(bsz):
        x2 = (y[b] - m2) * inv2 * g2[...] + b2[...]
        x2 = jnp.maximum(x2, 0.0)
        rows.append(jnp.sum(x2 * ow[...], axis=-1) + ob[0])
    out[...] = jnp.stack(rows)[None]


def kernel(data, emb, lin_W, att_i, att_j, att_em_i, att_em_j, gnn_bias,
           bn1_g, bn1_b, bn2_g, bn2_b, out_W, out_b):
    import functools

    bsz, n, f = data.shape
    d = emb.shape[1]
    cnt = float(bsz * n)
    nb = _pick_block(n, 1000)
    ib = _pick_block(n, 200)
    if ib == n and n % 8 == 0 and n > 8:
        ib = 8
    gn = n // nb
    gi = n // ib

    ai = att_i.reshape(1, d)
    aj = att_j.reshape(1, d)
    aei = att_em_i.reshape(1, d)
    aej = att_em_j.reshape(1, d)
    g1 = bn1_g.reshape(1, d)
    b1 = bn1_b.reshape(1, d)
    g2 = bn2_g.reshape(1, d)
    b2 = bn2_b.reshape(1, d)
    bias = gnn_bias.reshape(1, d)
    ow = out_W.reshape(1, d)

    f32 = jnp.float32
    xlin, s, t, invn = pl.pallas_call(
        _k1_body,
        grid=(gn,),
        in_specs=[
            pl.BlockSpec((bsz, nb, f), lambda i: (0, i, 0)),
            pl.BlockSpec((nb, d), lambda i: (i, 0)),
            pl.BlockSpec((f, d), lambda i: (0, 0)),
            pl.BlockSpec((1, d), lambda i: (0, 0)),
            pl.BlockSpec((1, d), lambda i: (0, 0)),
            pl.BlockSpec((1, d), lambda i: (0, 0)),
            pl.BlockSpec((1, d), lambda i: (0, 0)),
        ],
        out_specs=[
            pl.BlockSpec((bsz, nb, d), lambda i: (0, i, 0)),
            pl.BlockSpec((1, bsz, nb), lambda i: (i, 0, 0)),
            pl.BlockSpec((1, bsz, nb), lambda i: (i, 0, 0)),
            pl.BlockSpec((1, 1, nb), lambda i: (i, 0, 0)),
        ],
        out_shape=[
            jax.ShapeDtypeStruct((bsz, n, d), f32),
            jax.ShapeDtypeStruct((gn, bsz, nb), f32),
            jax.ShapeDtypeStruct((gn, bsz, nb), f32),
            jax.ShapeDtypeStruct((gn, 1, nb), f32),
        ],
    )(data, emb, lin_W, ai, aj, aei, aej)
    t = t.transpose(1, 0, 2).reshape(bsz, n)
    s = s.transpose(1, 0, 2).reshape(bsz, gi, ib).transpose(1, 0, 2)
    invn = invn.reshape(1, n)

    msgs, sums, sumsq = pl.pallas_call(
        functools.partial(_k2_body, ib=ib, n=n, topk=_TOPK),
        grid=(gi,),
        in_specs=[
            pl.BlockSpec((ib, d), lambda i: (i, 0)),
            pl.BlockSpec((n, d), lambda i: (0, 0)),
            pl.BlockSpec((1, n), lambda i: (0, 0)),
            pl.BlockSpec((1, bsz, ib), lambda i: (i, 0, 0)),
            pl.BlockSpec((bsz, n), lambda i: (0, 0)),
            pl.BlockSpec((bsz, n, d), lambda i: (0, 0, 0)),
        ],
        out_specs=[
            pl.BlockSpec((bsz, ib, d), lambda i: (0, i, 0)),
            pl.BlockSpec((1, d), lambda i: (0, 0)),
            pl.BlockSpec((1, d), lambda i: (0, 0)),
        ],
        out_shape=[
            jax.ShapeDtypeStruct((bsz, n, d), f32),
            jax.ShapeDtypeStruct((1, d), f32),
            jax.ShapeDtypeStruct((1, d), f32),
        ],
        scratch_shapes=[pltpu.VMEM((ib, n), f32)],
    )(emb, emb, invn, s, t, xlin)

    y, ys, yq = pl.pallas_call(
        functools.partial(_k3_body, cnt=cnt),
        grid=(gn,),
        in_specs=[
            pl.BlockSpec((bsz, nb, d), lambda i: (0, i, 0)),
            pl.BlockSpec((nb, d), lambda i: (i, 0)),
            pl.BlockSpec((1, d), lambda i: (0, 0)),
            pl.BlockSpec((1, d), lambda i: (0, 0)),
            pl.BlockSpec((1, d), lambda i: (0, 0)),
            pl.BlockSpec((1, d), lambda i: (0, 0)),
            pl.BlockSpec((1, d), lambda i: (0, 0)),
        ],
        out_specs=[
            pl.BlockSpec((bsz, nb, d), lambda i: (0, i, 0)),
            pl.BlockSpec((1, d), lambda i: (0, 0)),
            pl.BlockSpec((1, d), lambda i: (0, 0)),
        ],
        out_shape=[
            jax.ShapeDtypeStruct((bsz, n, d), f32),
            jax.ShapeDtypeStruct((1, d), f32),
            jax.ShapeDtypeStruct((1, d), f32),
        ],
    )(msgs, emb, bias, g1, b1, sums, sumsq)

    out = pl.pallas_call(
        functools.partial(_k4_body, cnt=cnt),
        grid=(gn,),
        in_specs=[
            pl.BlockSpec((bsz, nb, d), lambda i: (0, i, 0)),
            pl.BlockSpec((1, d), lambda i: (0, 0)),
            pl.BlockSpec((1, d), lambda i: (0, 0)),
            pl.BlockSpec((1, d), lambda i: (0, 0)),
            pl.BlockSpec((1, d), lambda i: (0, 0)),
            pl.BlockSpec((1, d), lambda i: (0, 0)),
            pl.BlockSpec(memory_space=pltpu.SMEM),
        ],
        out_specs=[pl.BlockSpec((1, bsz, nb), lambda i: (i, 0, 0))],
        out_shape=[jax.ShapeDtypeStruct((gn, bsz, nb), f32)],
    )(y, g2, b2, ys, yq, ow, out_b)[0]

    return out.transpose(1, 0, 2).reshape(bsz, n)


# speedup vs baseline: 33.9219x; 1.3693x over previous
import jax
import jax.numpy as jnp
from jax.experimental import pallas as pl
from jax.experimental.pallas import tpu as pltpu

_NEG_SLOPE = 0.2
_EPS = 1e-5
_TOPK = 32


def _pick_block(n, want):
    if n % want == 0:
        return want
    return n


def _k1_body(data, emb, linw, ai, aj, aei, aej, xlin, s, t, invn):
    e = emb[...]
    w = linw[...]
    bsz = data.shape[0]
    ones = jnp.ones((e.shape[0], 1), jnp.float32)
    xs, ss, ts = [], [], []
    for b in range(bsz):
        xb = jax.lax.dot_general(data[b], w, (((1,), (0,)), ((), ())))
        xs.append(jnp.concatenate([xb, ones], axis=1))
        ss.append(jnp.sum(xb * ai[...], axis=-1) + jnp.sum(e * aei[...], axis=-1))
        ts.append(jnp.sum(xb * aj[...], axis=-1) + jnp.sum(e * aej[...], axis=-1))
    xlin[...] = jnp.stack(xs)
    s[...] = jnp.stack(ss)[None]
    t[...] = jnp.stack(ts)[None]
    invn[...] = jax.lax.rsqrt(jnp.sum(e * e, axis=-1))[None, None, :]


def _k2_body(embi, emba, invn, s, t, xlin, msgs, sums, sumsq, vals, *, ib, n, topk):
    i0 = pl.program_id(0) * ib
    bsz = t.shape[0]
    score = jax.lax.dot_general(embi[...], emba[...], (((1,), (1,)), ((), ())))
    score = score * invn[...]
    vals[...] = score
    neg = jnp.float32(-jnp.inf)

    def extract(_, carry):
        v = vals[...]
        m = jnp.max(v, axis=1, keepdims=True)
        vals[...] = jnp.where(v == m, neg, v)
        return carry

    jax.lax.fori_loop(0, topk, extract, 0)
    topk_mask = vals[...] == neg
    rows = jax.lax.broadcasted_iota(jnp.int32, (ib, n), 0) + i0
    cols = jax.lax.broadcasted_iota(jnp.int32, (ib, n), 1)
    allowed = jnp.logical_or(topk_mask, rows == cols)

    ms = []
    for b in range(bsz):
        sb = s[0, b]
        alpha = sb[:, None] + t[b, :][None, :]
        alpha = jnp.maximum(alpha, _NEG_SLOPE * alpha)
        ex = jnp.where(allowed, jnp.exp(alpha), 0.0)
        ms.append(
            jax.lax.dot_general(
                ex, xlin[b], (((1,), (0,)), ((), ())),
                precision=jax.lax.Precision.HIGHEST,
            )
        )
    me = jnp.stack(ms)
    mall = me[:, :, :-1] / me[:, :, -1:]
    msgs[...] = mall
    ps = jnp.sum(jnp.sum(mall, axis=0), axis=0)[None, :]
    pq = jnp.sum(jnp.sum(mall * mall, axis=0), axis=0)[None, :]

    @pl.when(pl.program_id(0) == 0)
    def _init():
        sums[...] = jnp.zeros_like(sums)
        sumsq[...] = jnp.zeros_like(sumsq)

    sums[...] += ps
    sumsq[...] += pq


def _k3_body(msgs, emb, bias, g1, b1, sums, sumsq, y, ys, yq, *, cnt):
    m0 = sums[...] / cnt
    v1 = sumsq[...] / cnt - m0 * m0
    m1 = m0 + bias[...]
    inv1 = jax.lax.rsqrt(v1 + _EPS)
    e = emb[...]
    bsz = msgs.shape[0]
    outs = []
    for b in range(bsz):
        o = msgs[b] + bias[...]
        o = (o - m1) * inv1 * g1[...] + b1[...]
        o = jnp.maximum(o, 0.0)
        outs.append(o * e)
    yall = jnp.stack(outs)
    y[...] = yall
    ps = jnp.sum(jnp.sum(yall, axis=0), axis=0)[None, :]
    pq = jnp.sum(jnp.sum(yall * yall, axis=0), axis=0)[None, :]

    @pl.when(pl.program_id(0) == 0)
    def _init():
        ys[...] = jnp.zeros_like(ys)
        yq[...] = jnp.zeros_like(yq)

    ys[...] += ps
    yq[...] += pq


def _k4_body(y, g2, b2, ys, yq, ow, ob, out, *, cnt):
    m2 = ys[...] / cnt
    v2 = yq[...] / cnt - m2 * m2
    inv2 = jax.lax.rsqrt(v2 + _EPS)
    bsz = y.shape[0]
    rows = []
    for b in range(bsz):
        x2 = (y[b] - m2) * inv2 * g2[...] + b2[...]
        x2 = jnp.maximum(x2, 0.0)
        rows.append(jnp.sum(x2 * ow[...], axis=-1) + ob[0])
    out[...] = jnp.stack(rows)[None]


def kernel(data, emb, lin_W, att_i, att_j, att_em_i, att_em_j, gnn_bias,
           bn1_g, bn1_b, bn2_g, bn2_b, out_W, out_b):
    import functools

    bsz, n, f = data.shape
    d = emb.shape[1]
    cnt = float(bsz * n)
    nb = _pick_block(n, 1000)
    ib = _pick_block(n, 200)
    if ib == n and n % 8 == 0 and n > 8:
        ib = 8
    gn = n // nb
    gi = n // ib

    ai = att_i.reshape(1, d)
    aj = att_j.reshape(1, d)
    aei = att_em_i.reshape(1, d)
    aej = att_em_j.reshape(1, d)
    g1 = bn1_g.reshape(1, d)
    b1 = bn1_b.reshape(1, d)
    g2 = bn2_g.reshape(1, d)
    b2 = bn2_b.reshape(1, d)
    bias = gnn_bias.reshape(1, d)
    ow = out_W.reshape(1, d)

    f32 = jnp.float32
    xlin, s, t, invn = pl.pallas_call(
        _k1_body,
        grid=(gn,),
        in_specs=[
            pl.BlockSpec((bsz, nb, f), lambda i: (0, i, 0)),
            pl.BlockSpec((nb, d), lambda i: (i, 0)),
            pl.BlockSpec((f, d), lambda i: (0, 0)),
            pl.BlockSpec((1, d), lambda i: (0, 0)),
            pl.BlockSpec((1, d), lambda i: (0, 0)),
            pl.BlockSpec((1, d), lambda i: (0, 0)),
            pl.BlockSpec((1, d), lambda i: (0, 0)),
        ],
        out_specs=[
            pl.BlockSpec((bsz, nb, d + 1), lambda i: (0, i, 0)),
            pl.BlockSpec((1, bsz, nb), lambda i: (i, 0, 0)),
            pl.BlockSpec((1, bsz, nb), lambda i: (i, 0, 0)),
            pl.BlockSpec((1, 1, nb), lambda i: (i, 0, 0)),
        ],
        out_shape=[
            jax.ShapeDtypeStruct((bsz, n, d + 1), f32),
            jax.ShapeDtypeStruct((gn, bsz, nb), f32),
            jax.ShapeDtypeStruct((gn, bsz, nb), f32),
            jax.ShapeDtypeStruct((gn, 1, nb), f32),
        ],
    )(data, emb, lin_W, ai, aj, aei, aej)
    t = t.transpose(1, 0, 2).reshape(bsz, n)
    s = s.transpose(1, 0, 2).reshape(bsz, gi, ib).transpose(1, 0, 2)
    invn = invn.reshape(1, n)

    msgs, sums, sumsq = pl.pallas_call(
        functools.partial(_k2_body, ib=ib, n=n, topk=_TOPK),
        grid=(gi,),
        in_specs=[
            pl.BlockSpec((ib, d), lambda i: (i, 0)),
            pl.BlockSpec((n, d), lambda i: (0, 0)),
            pl.BlockSpec((1, n), lambda i: (0, 0)),
            pl.BlockSpec((1, bsz, ib), lambda i: (i, 0, 0)),
            pl.BlockSpec((bsz, n), lambda i: (0, 0)),
            pl.BlockSpec((bsz, n, d + 1), lambda i: (0, 0, 0)),
        ],
        out_specs=[
            pl.BlockSpec((bsz, ib, d), lambda i: (0, i, 0)),
            pl.BlockSpec((1, d), lambda i: (0, 0)),
            pl.BlockSpec((1, d), lambda i: (0, 0)),
        ],
        out_shape=[
            jax.ShapeDtypeStruct((bsz, n, d), f32),
            jax.ShapeDtypeStruct((1, d), f32),
            jax.ShapeDtypeStruct((1, d), f32),
        ],
        scratch_shapes=[pltpu.VMEM((ib, n), f32)],
    )(emb, emb, invn, s, t, xlin)

    y, ys, yq = pl.pallas_call(
        functools.partial(_k3_body, cnt=cnt),
        grid=(gn,),
        in_specs=[
            pl.BlockSpec((bsz, nb, d), lambda i: (0, i, 0)),
            pl.BlockSpec((nb, d), lambda i: (i, 0)),
            pl.BlockSpec((1, d), lambda i: (0, 0)),
            pl.BlockSpec((1, d), lambda i: (0, 0)),
            pl.BlockSpec((1, d), lambda i: (0, 0)),
            pl.BlockSpec((1, d), lambda i: (0, 0)),
            pl.BlockSpec((1, d), lambda i: (0, 0)),
        ],
        out_specs=[
            pl.BlockSpec((bsz, nb, d), lambda i: (0, i, 0)),
            pl.BlockSpec((1, d), lambda i: (0, 0)),
            pl.BlockSpec((1, d), lambda i: (0, 0)),
        ],
        out_shape=[
            jax.ShapeDtypeStruct((bsz, n, d), f32),
            jax.ShapeDtypeStruct((1, d), f32),
            jax.ShapeDtypeStruct((1, d), f32),
        ],
    )(msgs, emb, bias, g1, b1, sums, sumsq)

    out = pl.pallas_call(
        functools.partial(_k4_body, cnt=cnt),
        grid=(gn,),
        in_specs=[
            pl.BlockSpec((bsz, nb, d), lambda i: (0, i, 0)),
            pl.BlockSpec((1, d), lambda i: (0, 0)),
            pl.BlockSpec((1, d), lambda i: (0, 0)),
            pl.BlockSpec((1, d), lambda i: (0, 0)),
            pl.BlockSpec((1, d), lambda i: (0, 0)),
            pl.BlockSpec((1, d), lambda i: (0, 0)),
            pl.BlockSpec(memory_space=pltpu.SMEM),
        ],
        out_specs=[pl.BlockSpec((1, bsz, nb), lambda i: (i, 0, 0))],
        out_shape=[jax.ShapeDtypeStruct((gn, bsz, nb), f32)],
    )(y, g2, b2, ys, yq, ow, out_b)[0]

    return out.transpose(1, 0, 2).reshape(bsz, n)


# speedup vs baseline: 66.8693x; 1.9713x over previous
import jax
import jax.numpy as jnp
from jax.experimental import pallas as pl
from jax.experimental.pallas import tpu as pltpu

_NEG_SLOPE = 0.2
_EPS = 1e-5
_TOPK = 32


def _pick_block(n, want):
    if n % want == 0:
        return want
    return n


def _k1_body(data, emb, linw, ai, aj, aei, aej, xlin, s, t, invn):
    e = emb[...]
    w = linw[...]
    bsz = data.shape[0]
    ones = jnp.ones((e.shape[0], 1), jnp.float32)
    xs, ss, ts = [], [], []
    for b in range(bsz):
        xb = jax.lax.dot_general(data[b], w, (((1,), (0,)), ((), ())))
        xs.append(jnp.concatenate([xb, ones], axis=1))
        ss.append(jnp.sum(xb * ai[...], axis=-1) + jnp.sum(e * aei[...], axis=-1))
        ts.append(jnp.sum(xb * aj[...], axis=-1) + jnp.sum(e * aej[...], axis=-1))
    xlin[...] = jnp.stack(xs)
    s[...] = jnp.stack(ss)[None]
    t[...] = jnp.stack(ts)[None]
    invn[...] = jax.lax.rsqrt(jnp.sum(e * e, axis=-1))[None, None, :]


def _k2_body(embi, emba, invn, s, t, xlin, msgs, sums, sumsq, vals, *, ib, n, topk):
    i0 = pl.program_id(0) * ib
    bsz = t.shape[0]
    score = jax.lax.dot_general(embi[...], emba[...], (((1,), (1,)), ((), ())))
    score = score * invn[...]
    vals[...] = score
    neg = jnp.float32(-jnp.inf)

    def extract(_, carry):
        v = vals[...]
        m = jnp.max(v, axis=1, keepdims=True)
        vals[...] = jnp.where(v == m, neg, v)
        return carry

    jax.lax.fori_loop(0, topk, extract, 0)
    topk_mask = vals[...] == neg
    rows = jax.lax.broadcasted_iota(jnp.int32, (ib, n), 0) + i0
    cols = jax.lax.broadcasted_iota(jnp.int32, (ib, n), 1)
    allowed = jnp.logical_or(topk_mask, rows == cols)

    ms = []
    for b in range(bsz):
        sb = s[0, b]
        alpha = sb[:, None] + t[b, :][None, :]
        alpha = jnp.maximum(alpha, _NEG_SLOPE * alpha)
        ex = jnp.where(allowed, jnp.exp(alpha), 0.0)
        ms.append(
            jax.lax.dot_general(
                ex, xlin[b], (((1,), (0,)), ((), ())),
                precision=jax.lax.Precision.DEFAULT,
            )
        )
    me = jnp.stack(ms)
    mall = me[:, :, :-1] / me[:, :, -1:]
    msgs[...] = mall
    ps = jnp.sum(jnp.sum(mall, axis=0), axis=0)[None, :]
    pq = jnp.sum(jnp.sum(mall * mall, axis=0), axis=0)[None, :]

    @pl.when(pl.program_id(0) == 0)
    def _init():
        sums[...] = jnp.zeros_like(sums)
        sumsq[...] = jnp.zeros_like(sumsq)

    sums[...] += ps
    sumsq[...] += pq


def _k3_body(msgs, emb, bias, g1, b1, sums, sumsq, y, ys, yq, *, cnt):
    m0 = sums[...] / cnt
    v1 = sumsq[...] / cnt - m0 * m0
    m1 = m0 + bias[...]
    inv1 = jax.lax.rsqrt(v1 + _EPS)
    e = emb[...]
    bsz = msgs.shape[0]
    outs = []
    for b in range(bsz):
        o = msgs[b] + bias[...]
        o = (o - m1) * inv1 * g1[...] + b1[...]
        o = jnp.maximum(o, 0.0)
        outs.append(o * e)
    yall = jnp.stack(outs)
    y[...] = yall
    ps = jnp.sum(jnp.sum(yall, axis=0), axis=0)[None, :]
    pq = jnp.sum(jnp.sum(yall * yall, axis=0), axis=0)[None, :]

    @pl.when(pl.program_id(0) == 0)
    def _init():
        ys[...] = jnp.zeros_like(ys)
        yq[...] = jnp.zeros_like(yq)

    ys[...] += ps
    yq[...] += pq


def _k4_body(y, g2, b2, ys, yq, ow, ob, out, *, cnt):
    m2 = ys[...] / cnt
    v2 = yq[...] / cnt - m2 * m2
    inv2 = jax.lax.rsqrt(v2 + _EPS)
    bsz = y.shape[0]
    rows = []
    for b in range(bsz):
        x2 = (y[b] - m2) * inv2 * g2[...] + b2[...]
        x2 = jnp.maximum(x2, 0.0)
        rows.append(jnp.sum(x2 * ow[...], axis=-1) + ob[0])
    out[...] = jnp.stack(rows)[None]


def kernel(data, emb, lin_W, att_i, att_j, att_em_i, att_em_j, gnn_bias,
           bn1_g, bn1_b, bn2_g, bn2_b, out_W, out_b):
    import functools

    bsz, n, f = data.shape
    d = emb.shape[1]
    cnt = float(bsz * n)
    nb = _pick_block(n, 1000)
    ib = _pick_block(n, 200)
    if ib == n and n % 8 == 0 and n > 8:
        ib = 8
    gn = n // nb
    gi = n // ib

    ai = att_i.reshape(1, d)
    aj = att_j.reshape(1, d)
    aei = att_em_i.reshape(1, d)
    aej = att_em_j.reshape(1, d)
    g1 = bn1_g.reshape(1, d)
    b1 = bn1_b.reshape(1, d)
    g2 = bn2_g.reshape(1, d)
    b2 = bn2_b.reshape(1, d)
    bias = gnn_bias.reshape(1, d)
    ow = out_W.reshape(1, d)

    f32 = jnp.float32
    xlin, s, t, invn = pl.pallas_call(
        _k1_body,
        grid=(gn,),
        in_specs=[
            pl.BlockSpec((bsz, nb, f), lambda i: (0, i, 0)),
            pl.BlockSpec((nb, d), lambda i: (i, 0)),
            pl.BlockSpec((f, d), lambda i: (0, 0)),
            pl.BlockSpec((1, d), lambda i: (0, 0)),
            pl.BlockSpec((1, d), lambda i: (0, 0)),
            pl.BlockSpec((1, d), lambda i: (0, 0)),
            pl.BlockSpec((1, d), lambda i: (0, 0)),
        ],
        out_specs=[
            pl.BlockSpec((bsz, nb, d + 1), lambda i: (0, i, 0)),
            pl.BlockSpec((1, bsz, nb), lambda i: (i, 0, 0)),
            pl.BlockSpec((1, bsz, nb), lambda i: (i, 0, 0)),
            pl.BlockSpec((1, 1, nb), lambda i: (i, 0, 0)),
        ],
        out_shape=[
            jax.ShapeDtypeStruct((bsz, n, d + 1), f32),
            jax.ShapeDtypeStruct((gn, bsz, nb), f32),
            jax.ShapeDtypeStruct((gn, bsz, nb), f32),
            jax.ShapeDtypeStruct((gn, 1, nb), f32),
        ],
    )(data, emb, lin_W, ai, aj, aei, aej)
    t = t.transpose(1, 0, 2).reshape(bsz, n)
    s = s.transpose(1, 0, 2).reshape(bsz, gi, ib).transpose(1, 0, 2)
    invn = invn.reshape(1, n)

    msgs, sums, sumsq = pl.pallas_call(
        functools.partial(_k2_body, ib=ib, n=n, topk=_TOPK),
        grid=(gi,),
        in_specs=[
            pl.BlockSpec((ib, d), lambda i: (i, 0)),
            pl.BlockSpec((n, d), lambda i: (0, 0)),
            pl.BlockSpec((1, n), lambda i: (0, 0)),
            pl.BlockSpec((1, bsz, ib), lambda i: (i, 0, 0)),
            pl.BlockSpec((bsz, n), lambda i: (0, 0)),
            pl.BlockSpec((bsz, n, d + 1), lambda i: (0, 0, 0)),
        ],
        out_specs=[
            pl.BlockSpec((bsz, ib, d), lambda i: (0, i, 0)),
            pl.BlockSpec((1, d), lambda i: (0, 0)),
            pl.BlockSpec((1, d), lambda i: (0, 0)),
        ],
        out_shape=[
            jax.ShapeDtypeStruct((bsz, n, d), f32),
            jax.ShapeDtypeStruct((1, d), f32),
            jax.ShapeDtypeStruct((1, d), f32),
        ],
        scratch_shapes=[pltpu.VMEM((ib, n), f32)],
    )(emb, emb, invn, s, t, xlin)

    y, ys, yq = pl.pallas_call(
        functools.partial(_k3_body, cnt=cnt),
        grid=(gn,),
        in_specs=[
            pl.BlockSpec((bsz, nb, d), lambda i: (0, i, 0)),
            pl.BlockSpec((nb, d), lambda i: (i, 0)),
            pl.BlockSpec((1, d), lambda i: (0, 0)),
            pl.BlockSpec((1, d), lambda i: (0, 0)),
            pl.BlockSpec((1, d), lambda i: (0, 0)),
            pl.BlockSpec((1, d), lambda i: (0, 0)),
            pl.BlockSpec((1, d), lambda i: (0, 0)),
        ],
        out_specs=[
            pl.BlockSpec((bsz, nb, d), lambda i: (0, i, 0)),
            pl.BlockSpec((1, d), lambda i: (0, 0)),
            pl.BlockSpec((1, d), lambda i: (0, 0)),
        ],
        out_shape=[
            jax.ShapeDtypeStruct((bsz, n, d), f32),
            jax.ShapeDtypeStruct((1, d), f32),
            jax.ShapeDtypeStruct((1, d), f32),
        ],
    )(msgs, emb, bias, g1, b1, sums, sumsq)

    out = pl.pallas_call(
        functools.partial(_k4_body, cnt=cnt),
        grid=(gn,),
        in_specs=[
            pl.BlockSpec((bsz, nb, d), lambda i: (0, i, 0)),
            pl.BlockSpec((1, d), lambda i: (0, 0)),
            pl.BlockSpec((1, d), lambda i: (0, 0)),
            pl.BlockSpec((1, d), lambda i: (0, 0)),
            pl.BlockSpec((1, d), lambda i: (0, 0)),
            pl.BlockSpec((1, d), lambda i: (0, 0)),
            pl.BlockSpec(memory_space=pltpu.SMEM),
        ],
        out_specs=[pl.BlockSpec((1, bsz, nb), lambda i: (i, 0, 0))],
        out_shape=[jax.ShapeDtypeStruct((gn, bsz, nb), f32)],
    )(y, g2, b2, ys, yq, ow, out_b)[0]

    return out.transpose(1, 0, 2).reshape(bsz, n)
